# baseline (speedup 1.0000x reference)
import jax
import jax.numpy as jnp
from jax import lax
from jax.experimental.layout import Format, Layout
from jax.experimental import pallas as pl
from jax.experimental.pallas import tpu as pltpu
from jax.experimental.pallas import tpu_sc as plsc

B = 16384
T = 200
D = 32
GSPLIT = ((0, 96), (96, 104))
NW = 32
SEQ_W = B // NW
CSEQ = 8
NCH = SEQ_W // CSEQ
GPC = CSEQ * len(GSPLIT)
IDXC = CSEQ * T
NVF = T // 16


def _sc_body(x2_hbm, table_hbm, out_hbm,
             idx_v0, idx_v1, rows_v0, rows_v1, den_v, out_v,
             sem_g0, sem_g1, sem_i0, sem_i1):
    wid = lax.axis_index("s") * 2 + lax.axis_index("c")
    seq0 = wid * SEQ_W
    zf = jnp.zeros((16,), jnp.float32)
    zi = jnp.zeros((16,), jnp.int32)
    onef = jnp.ones((16,), jnp.float32)
    lanes = jnp.arange(16, dtype=jnp.int32)
    idx_bufs = (idx_v0, idx_v1)
    rows_bufs = (rows_v0, rows_v1)
    sems_g = (sem_g0, sem_g1)
    sems_i = (sem_i0, sem_i1)

    def idx_src(c):
        return x2_hbm.at[pl.ds(seq0 + c * CSEQ, CSEQ), :]

    def idx_dst(b):
        return idx_bufs[b]

    def fire_gathers(b):
        for s in range(CSEQ):
            for off, sz in GSPLIT:
                pltpu.async_copy(
                    table_hbm.at[idx_bufs[b].at[s, pl.ds(off, sz)]],
                    rows_bufs[b].at[pl.ds(s * T + off, sz), :],
                    sems_g[b],
                )

    def drain_gathers(b):
        for s in range(CSEQ):
            for off, sz in GSPLIT:
                pltpu.make_async_copy(
                    table_hbm.at[idx_bufs[b].at[s, pl.ds(off, sz)]],
                    rows_bufs[b].at[pl.ds(s * T + off, sz), :],
                    sems_g[b],
                ).wait()

    def count_chunk(b):
        idxb = idx_bufs[b]
        for s in range(CSEQ):
            cnt = zi
            for t in range(NVF):
                v = idxb[s, pl.ds(t * 16, 16)]
                cnt = cnt + plsc.all_reduce_population_count(v != 0)
            v = idxb[s, pl.ds(T - 16, 16)]
            cnt = cnt + plsc.all_reduce_population_count(
                (v != 0) & (lanes >= 8))
            den_v[pl.ds((b * CSEQ + s) * 16, 16)] = jnp.maximum(
                cnt.astype(jnp.float32), onef)

    def compute_chunk(c, b):
        rowsb = rows_bufs[b]
        for s in range(CSEQ):
            r0 = s * T

            def acc_body(t, carry, _r0=r0, _rowsb=rowsb):
                a0, a1 = carry
                return (a0 + _rowsb[_r0 + t, pl.ds(0, 16)],
                        a1 + _rowsb[_r0 + t, pl.ds(16, 16)])

            a0, a1 = pl.loop(0, T, init_carry=(zf, zf), unroll=8)(acc_body)
            denom = den_v[pl.ds((b * CSEQ + s) * 16, 16)]
            lidx = c * CSEQ + s
            out_v[lidx, pl.ds(0, 16)] = a0 / denom
            out_v[lidx, pl.ds(16, 16)] = a1 / denom

    pltpu.sync_copy(idx_src(0), idx_dst(0))
    pltpu.async_copy(idx_src(1), idx_dst(1), sem_i1)
    count_chunk(0)
    fire_gathers(0)

    @pl.loop(0, NCH, step=2)
    def _iter(c0):
        for k in range(2):
            c = c0 + k
            b, nb = k, 1 - k
            drain_gathers(b)

            @pl.when(c + 1 < NCH)
            def _stage():
                pltpu.make_async_copy(idx_src(c + 1), idx_dst(nb),
                                      sems_i[nb]).wait()
                count_chunk(nb)
                fire_gathers(nb)

            @pl.when(c + 2 < NCH)
            def _prefetch():
                pltpu.async_copy(idx_src(c + 2), idx_dst(b), sems_i[b])

            compute_chunk(c, b)

    pltpu.sync_copy(out_v, out_hbm.at[pl.ds(seq0, SEQ_W), :])


_TBW = 2048


def _tc_transpose_body(tt_ref, o_ref):
    o_ref[...] = tt_ref[...].T


def _tc_transpose(tt):
    v = tt.shape[1]
    return pl.pallas_call(
        _tc_transpose_body,
        grid=(v // _TBW,),
        in_specs=[pl.BlockSpec((D, _TBW), lambda i: (0, i))],
        out_specs=pl.BlockSpec((_TBW, D), lambda i: (i, 0)),
        out_shape=jax.ShapeDtypeStruct((v, D), jnp.float32),
    )(tt)


@jax.jit
def kernel(x, table):
    run = pl.kernel(
        _sc_body,
        out_type=jax.ShapeDtypeStruct((B, D), jnp.float32),
        mesh=plsc.VectorSubcoreMesh(core_axis_name="c", subcore_axis_name="s"),
        compiler_params=pltpu.CompilerParams(
            use_tc_tiling_on_sc=False, needs_layout_passes=False),
        scratch_types=[
            pltpu.VMEM((CSEQ, T), jnp.int32),
            pltpu.VMEM((CSEQ, T), jnp.int32),
            pltpu.VMEM((IDXC, D), jnp.float32),
            pltpu.VMEM((IDXC, D), jnp.float32),
            pltpu.VMEM((2 * CSEQ * 16,), jnp.float32),
            pltpu.VMEM((SEQ_W, D), jnp.float32),
            pltpu.SemaphoreType.DMA,
            pltpu.SemaphoreType.DMA,
            pltpu.SemaphoreType.DMA,
            pltpu.SemaphoreType.DMA,
        ],
    )
    table_rm = _tc_transpose(table.T)
    return run(x, table_rm)


# speedup vs baseline: 1.3948x; 1.3948x over previous
import jax
import jax.numpy as jnp
from jax import lax
from jax.experimental import pallas as pl
from jax.experimental.pallas import tpu as pltpu
from jax.experimental.pallas import tpu_sc as plsc

B = 16384
T = 200
D = 32
GSPLIT = ((0, 96), (96, 104))
NW = 32
SEQ_W = B // NW
CSEQ = 8
NCH = SEQ_W // CSEQ
GPC = CSEQ * len(GSPLIT)
IDXC = CSEQ * T
NVF = T // 16


def _sc_body(x2_hbm, table_hbm, out_hbm,
             idx_v0, idx_v1, rows_v0, rows_v1, den_v, out_v,
             sem_g0, sem_g1, sem_i0, sem_i1):
    wid = lax.axis_index("s") * 2 + lax.axis_index("c")
    seq0 = wid * SEQ_W
    zf = jnp.zeros((16,), jnp.float32)
    zi = jnp.zeros((16,), jnp.int32)
    onef = jnp.ones((16,), jnp.float32)
    lanes = jnp.arange(16, dtype=jnp.int32)
    idx_bufs = (idx_v0, idx_v1)
    rows_bufs = (rows_v0, rows_v1)
    sems_g = (sem_g0, sem_g1)
    sems_i = (sem_i0, sem_i1)

    def idx_src(c):
        return x2_hbm.at[pl.ds(seq0 + c * CSEQ, CSEQ), :]

    def idx_dst(b):
        return idx_bufs[b]

    def fire_gathers(b):
        for s in range(CSEQ):
            for off, sz in GSPLIT:
                pltpu.async_copy(
                    table_hbm.at[idx_bufs[b].at[s, pl.ds(off, sz)]],
                    rows_bufs[b].at[pl.ds(s * T + off, sz), :],
                    sems_g[b],
                )

    def drain_gathers(b):
        for s in range(CSEQ):
            for off, sz in GSPLIT:
                pltpu.make_async_copy(
                    table_hbm.at[idx_bufs[b].at[s, pl.ds(off, sz)]],
                    rows_bufs[b].at[pl.ds(s * T + off, sz), :],
                    sems_g[b],
                ).wait()

    def count_chunk(b):
        idxb = idx_bufs[b]
        for s in range(CSEQ):
            cnt = zi
            for t in range(NVF):
                v = idxb[s, pl.ds(t * 16, 16)]
                cnt = cnt + plsc.all_reduce_population_count(v != 0)
            v = idxb[s, pl.ds(T - 16, 16)]
            cnt = cnt + plsc.all_reduce_population_count(
                (v != 0) & (lanes >= 8))
            den_v[pl.ds((b * CSEQ + s) * 16, 16)] = jnp.maximum(
                cnt.astype(jnp.float32), onef)

    def compute_chunk(c, b):
        rowsb = rows_bufs[b]
        for s in range(CSEQ):
            r0 = s * T

            def acc_body(t, carry, _r0=r0, _rowsb=rowsb):
                a0, a1 = carry
                return (a0 + _rowsb[_r0 + t, pl.ds(0, 16)],
                        a1 + _rowsb[_r0 + t, pl.ds(16, 16)])

            a0, a1 = pl.loop(0, T, init_carry=(zf, zf), unroll=8)(acc_body)
            denom = den_v[pl.ds((b * CSEQ + s) * 16, 16)]
            lidx = c * CSEQ + s
            out_v[lidx, pl.ds(0, 16)] = a0 / denom
            out_v[lidx, pl.ds(16, 16)] = a1 / denom

    pltpu.sync_copy(idx_src(0), idx_dst(0))
    pltpu.async_copy(idx_src(1), idx_dst(1), sem_i1)
    count_chunk(0)
    fire_gathers(0)

    @pl.loop(0, NCH, step=2)
    def _iter(c0):
        for k in range(2):
            c = c0 + k
            b, nb = k, 1 - k
            drain_gathers(b)

            @pl.when(c + 1 < NCH)
            def _stage():
                pltpu.make_async_copy(idx_src(c + 1), idx_dst(nb),
                                      sems_i[nb]).wait()
                count_chunk(nb)
                fire_gathers(nb)

            @pl.when(c + 2 < NCH)
            def _prefetch():
                pltpu.async_copy(idx_src(c + 2), idx_dst(b), sems_i[b])

            compute_chunk(c, b)

    pltpu.sync_copy(out_v, out_hbm.at[pl.ds(seq0, SEQ_W), :])


@jax.jit
def kernel(x, table):
    run = pl.kernel(
        _sc_body,
        out_type=jax.ShapeDtypeStruct((B, D), jnp.float32),
        mesh=plsc.VectorSubcoreMesh(core_axis_name="c", subcore_axis_name="s"),
        compiler_params=pltpu.CompilerParams(
            use_tc_tiling_on_sc=False, needs_layout_passes=False),
        scratch_types=[
            pltpu.VMEM((CSEQ, T), jnp.int32),
            pltpu.VMEM((CSEQ, T), jnp.int32),
            pltpu.VMEM((IDXC, D), jnp.float32),
            pltpu.VMEM((IDXC, D), jnp.float32),
            pltpu.VMEM((2 * CSEQ * 16,), jnp.float32),
            pltpu.VMEM((SEQ_W, D), jnp.float32),
            pltpu.SemaphoreType.DMA,
            pltpu.SemaphoreType.DMA,
            pltpu.SemaphoreType.DMA,
            pltpu.SemaphoreType.DMA,
        ],
    )
    return run(x, table)
